# baseline (speedup 1.0000x reference)
import functools
import math

import jax
import jax.numpy as jnp
from jax.experimental import pallas as pl
from jax.experimental.pallas import tpu as pltpu

_SORT8 = [(0, 1), (2, 3), (4, 5), (6, 7),
          (0, 2), (1, 3), (4, 6), (5, 7),
          (1, 2), (5, 6), (0, 4), (3, 7),
          (1, 5), (2, 6),
          (1, 4), (3, 6),
          (2, 4), (3, 5),
          (3, 4)]


def _fused_body(q_ref, k_ref, v_ref, sal_ref, w1_ref, b1_ref, w2_ref, b2_ref,
                o_ref, x_scr, rw_scr, st_scr, w_scr, *, S_b, NS, L, inv_sqrt_d):
    l = pl.program_id(1)
    p = pl.program_id(2)
    ns = pl.program_id(3)

    @pl.when(jnp.logical_and(jnp.logical_and(l == 0, p == 0), ns == 0))
    def _router():
        h = jax.lax.dot_general(q_ref[...], w1_ref[...], (((1,), (0,)), ((), ())),
                                preferred_element_type=jnp.float32)
        h = jnp.maximum(h + b1_ref[...], 0.0)
        logits = jax.lax.dot_general(h.astype(jnp.bfloat16), w2_ref[...],
                                     (((1,), (0,)), ((), ())),
                                     preferred_element_type=jnp.float32)
        logits = logits + b2_ref[...]
        mx = jnp.max(logits, axis=1, keepdims=True)
        e = jnp.exp(logits - mx)
        rw_scr[...] = e / jnp.sum(e, axis=1, keepdims=True)

    @pl.when(jnp.logical_and(p == 0, ns < NS))
    def _scores():
        s = jax.lax.dot_general(q_ref[...], k_ref[0], (((1,), (1,)), ((), ())),
                                preferred_element_type=jnp.float32)
        x_scr[:, pl.ds(ns * S_b, S_b)] = s + sal_ref[0, 0, pl.ds(ns * S_b, S_b)][None, :]

    @pl.when(jnp.logical_and(p == 1, ns == 0))
    def _topk_stats():
        def resort(c):
            n = len(c)
            if n == 1:
                return c
            h = n // 2
            hi = [jnp.maximum(c[i], c[i + h]) for i in range(h)]
            lo = [jnp.minimum(c[i], c[i + h]) for i in range(h)]
            return resort(hi) + resort(lo)

        Wseg = (S_b * NS) // 8
        s8 = [x_scr[:, j * Wseg:(j + 1) * Wseg] for j in range(8)]
        for i, j in _SORT8:
            hi = jnp.maximum(s8[i], s8[j])
            s8[j] = jnp.minimum(s8[i], s8[j])
            s8[i] = hi
        w_cur = Wseg
        while w_cur > 1:
            h = w_cur // 2
            s8 = [jnp.maximum(s8[i][:, :h], s8[7 - i][:, h:]) for i in range(8)]
            if h > 1:
                s8 = resort(s8)
            w_cur = h
        t0 = s8[0]
        theta = s8[0]
        for i in range(1, 8):
            t0 = jnp.maximum(t0, s8[i])
            theta = jnp.minimum(theta, s8[i])
        denom = jnp.zeros_like(t0)
        for i in range(8):
            denom = denom + jnp.exp((s8[i] - t0) * inv_sqrt_d)
        lane = jax.lax.broadcasted_iota(jnp.int32, (1, L), 1)
        rw_l = jnp.sum(rw_scr[...] * (lane == l).astype(jnp.float32), axis=1,
                       keepdims=True)
        st_scr[:, 0:1] = t0 - jnp.log(rw_l / denom) / inv_sqrt_d
        st_scr[:, 1:2] = theta

    @pl.when(jnp.logical_and(p == 1, ns < NS))
    def _build_w():
        xt = x_scr[:, pl.ds(ns * S_b, S_b)]
        m = st_scr[:, 0:1]
        theta = st_scr[:, 1:2]
        w = jnp.where(xt >= theta, jnp.exp((xt - m) * inv_sqrt_d), 0.0)
        w_scr[ns % 2] = w.astype(jnp.bfloat16)

    @pl.when(jnp.logical_and(p == 1, ns >= 1))
    def _read():
        r = jax.lax.dot_general(w_scr[(ns - 1) % 2], v_ref[0],
                                (((1,), (0,)), ((), ())),
                                preferred_element_type=jnp.float32)

        @pl.when(jnp.logical_and(l == 0, ns == 1))
        def _set():
            o_ref[...] = r

        @pl.when(jnp.logical_not(jnp.logical_and(l == 0, ns == 1)))
        def _acc():
            o_ref[...] += r


def kernel(q, K, V, salience, W1, b1, W2, b2):
    B, T, D = q.shape
    L, S, _ = K.shape
    H = W1.shape[1]
    BT = B * T
    TM = 1024 if BT % 1024 == 0 else BT
    S_b = 1024 if S % 1024 == 0 else S
    NS = S // S_b

    qb = q.reshape(BT, D).astype(jnp.bfloat16)
    Kb = K.astype(jnp.bfloat16)
    Vb = V.astype(jnp.bfloat16)
    sal3 = (salience * math.sqrt(D)).reshape(L, 1, S)
    W1b = W1.astype(jnp.bfloat16)
    W2b = W2.astype(jnp.bfloat16)
    b1r = b1.reshape(1, H)
    b2r = b2.reshape(1, L)

    body = functools.partial(_fused_body, S_b=S_b, NS=NS, L=L,
                             inv_sqrt_d=float(1.0 / math.sqrt(D)))
    out = pl.pallas_call(
        body,
        grid=(BT // TM, L, 2, NS + 1),
        in_specs=[
            pl.BlockSpec((TM, D), lambda bt, l, p, ns: (bt, 0)),
            pl.BlockSpec((1, S_b, D),
                         lambda bt, l, p, ns: (l, jnp.where(p == 0, jnp.minimum(ns, NS - 1), NS - 1), 0)),
            pl.BlockSpec((1, S_b, D),
                         lambda bt, l, p, ns: (l, jnp.where(p == 1, jnp.clip(ns - 1, 0, NS - 1), 0), 0)),
            pl.BlockSpec((1, 1, S), lambda bt, l, p, ns: (l, 0, 0)),
            pl.BlockSpec((D, H), lambda bt, l, p, ns: (0, 0)),
            pl.BlockSpec((1, H), lambda bt, l, p, ns: (0, 0)),
            pl.BlockSpec((H, L), lambda bt, l, p, ns: (0, 0)),
            pl.BlockSpec((1, L), lambda bt, l, p, ns: (0, 0)),
        ],
        out_specs=pl.BlockSpec((TM, D), lambda bt, l, p, ns: (bt, 0)),
        out_shape=jax.ShapeDtypeStruct((BT, D), jnp.float32),
        scratch_shapes=[
            pltpu.VMEM((TM, S), jnp.float32),
            pltpu.VMEM((TM, L), jnp.float32),
            pltpu.VMEM((TM, 8), jnp.float32),
            pltpu.VMEM((2, TM, S_b), jnp.bfloat16),
        ],
        compiler_params=pltpu.CompilerParams(
            dimension_semantics=("arbitrary",) * 4,
            vmem_limit_bytes=62 * 1024 * 1024,
        ),
    )(qb, Kb, Vb, sal3, W1b, b1r, W2b, b2r)
    return out.reshape(B, T, D)


# speedup vs baseline: 1.0840x; 1.0840x over previous
import functools
import math

import jax
import jax.numpy as jnp
from jax.experimental import pallas as pl
from jax.experimental.pallas import tpu as pltpu

_SORT8 = [(0, 1), (2, 3), (4, 5), (6, 7),
          (0, 2), (1, 3), (4, 6), (5, 7),
          (1, 2), (5, 6), (0, 4), (3, 7),
          (1, 5), (2, 6),
          (1, 4), (3, 6),
          (2, 4), (3, 5),
          (3, 4)]


def _fused_body(q_ref, k_ref, v_ref, sal_ref, w1_ref, b1_ref, w2_ref, b2_ref,
                o_ref, x_scr, rw_scr, st_scr, *, S_b, NS, L, inv_sqrt_d):
    l = pl.program_id(1)
    p = pl.program_id(2)
    ns = pl.program_id(3)

    @pl.when(jnp.logical_and(jnp.logical_and(l == 0, p == 0), ns == 0))
    def _router():
        h = jax.lax.dot_general(q_ref[...], w1_ref[...], (((1,), (0,)), ((), ())),
                                preferred_element_type=jnp.float32)
        h = jnp.maximum(h + b1_ref[...], 0.0)
        logits = jax.lax.dot_general(h.astype(jnp.bfloat16), w2_ref[...],
                                     (((1,), (0,)), ((), ())),
                                     preferred_element_type=jnp.float32)
        logits = logits + b2_ref[...]
        mx = jnp.max(logits, axis=1, keepdims=True)
        e = jnp.exp(logits - mx)
        rw_scr[...] = e / jnp.sum(e, axis=1, keepdims=True)

    @pl.when(jnp.logical_and(p == 0, ns < NS))
    def _scores():
        s = jax.lax.dot_general(q_ref[...], k_ref[0], (((1,), (1,)), ((), ())),
                                preferred_element_type=jnp.float32)
        x_scr[:, pl.ds(ns * S_b, S_b)] = s + sal_ref[0, 0, pl.ds(ns * S_b, S_b)][None, :]

    @pl.when(jnp.logical_and(p == 1, ns == 0))
    def _topk_stats():
        def resort(c):
            n = len(c)
            if n == 1:
                return c
            h = n // 2
            hi = [jnp.maximum(c[i], c[i + h]) for i in range(h)]
            lo = [jnp.minimum(c[i], c[i + h]) for i in range(h)]
            return resort(hi) + resort(lo)

        Wseg = (S_b * NS) // 8
        s8 = [x_scr[:, j * Wseg:(j + 1) * Wseg] for j in range(8)]
        for i, j in _SORT8:
            hi = jnp.maximum(s8[i], s8[j])
            s8[j] = jnp.minimum(s8[i], s8[j])
            s8[i] = hi
        w_cur = Wseg
        while w_cur > 1:
            h = w_cur // 2
            s8 = [jnp.maximum(s8[i][:, :h], s8[7 - i][:, h:]) for i in range(8)]
            if h > 1:
                s8 = resort(s8)
            w_cur = h
        t0 = s8[0]
        theta = s8[0]
        for i in range(1, 8):
            t0 = jnp.maximum(t0, s8[i])
            theta = jnp.minimum(theta, s8[i])
        denom = jnp.zeros_like(t0)
        for i in range(8):
            denom = denom + jnp.exp((s8[i] - t0) * inv_sqrt_d)
        lane = jax.lax.broadcasted_iota(jnp.int32, (1, L), 1)
        rw_l = jnp.sum(rw_scr[...] * (lane == l).astype(jnp.float32), axis=1,
                       keepdims=True)
        st_scr[:, 0:1] = t0 - jnp.log(rw_l / denom) / inv_sqrt_d
        st_scr[:, 1:2] = theta

    @pl.when(p == 1)
    def _read():
        xt = x_scr[:, pl.ds(ns * S_b, S_b)]
        m = st_scr[:, 0:1]
        theta = st_scr[:, 1:2]
        w = jnp.where(xt >= theta, jnp.exp((xt - m) * inv_sqrt_d), 0.0)
        r = jax.lax.dot_general(w.astype(jnp.bfloat16), v_ref[0],
                                (((1,), (0,)), ((), ())),
                                preferred_element_type=jnp.float32)

        @pl.when(jnp.logical_and(l == 0, ns == 0))
        def _set():
            o_ref[...] = r

        @pl.when(jnp.logical_not(jnp.logical_and(l == 0, ns == 0)))
        def _acc():
            o_ref[...] += r


def kernel(q, K, V, salience, W1, b1, W2, b2):
    B, T, D = q.shape
    L, S, _ = K.shape
    H = W1.shape[1]
    BT = B * T
    TM = 1024 if BT % 1024 == 0 else BT
    S_b = 1024 if S % 1024 == 0 else S
    NS = S // S_b

    qb = q.reshape(BT, D).astype(jnp.bfloat16)
    Kb = K.astype(jnp.bfloat16)
    Vb = V.astype(jnp.bfloat16)
    sal3 = (salience * math.sqrt(D)).reshape(L, 1, S)
    W1b = W1.astype(jnp.bfloat16)
    W2b = W2.astype(jnp.bfloat16)
    b1r = b1.reshape(1, H)
    b2r = b2.reshape(1, L)

    body = functools.partial(_fused_body, S_b=S_b, NS=NS, L=L,
                             inv_sqrt_d=float(1.0 / math.sqrt(D)))
    out = pl.pallas_call(
        body,
        grid=(BT // TM, L, 2, NS),
        in_specs=[
            pl.BlockSpec((TM, D), lambda bt, l, p, ns: (bt, 0)),
            pl.BlockSpec((1, S_b, D),
                         lambda bt, l, p, ns: (l, jnp.where(p == 0, ns, NS - 1), 0)),
            pl.BlockSpec((1, S_b, D),
                         lambda bt, l, p, ns: (l, jnp.where(p == 1, ns, 0), 0)),
            pl.BlockSpec((1, 1, S), lambda bt, l, p, ns: (l, 0, 0)),
            pl.BlockSpec((D, H), lambda bt, l, p, ns: (0, 0)),
            pl.BlockSpec((1, H), lambda bt, l, p, ns: (0, 0)),
            pl.BlockSpec((H, L), lambda bt, l, p, ns: (0, 0)),
            pl.BlockSpec((1, L), lambda bt, l, p, ns: (0, 0)),
        ],
        out_specs=pl.BlockSpec((TM, D), lambda bt, l, p, ns: (bt, 0)),
        out_shape=jax.ShapeDtypeStruct((BT, D), jnp.float32),
        scratch_shapes=[
            pltpu.VMEM((TM, S), jnp.float32),
            pltpu.VMEM((TM, L), jnp.float32),
            pltpu.VMEM((TM, 8), jnp.float32),
        ],
        compiler_params=pltpu.CompilerParams(
            dimension_semantics=("arbitrary",) * 4,
            vmem_limit_bytes=62 * 1024 * 1024,
        ),
    )(qb, Kb, Vb, sal3, W1b, b1r, W2b, b2r)
    return out.reshape(B, T, D)


# speedup vs baseline: 1.1638x; 1.0736x over previous
import functools
import math

import jax
import jax.numpy as jnp
from jax.experimental import pallas as pl
from jax.experimental.pallas import tpu as pltpu

_SORT8 = [(0, 1), (2, 3), (4, 5), (6, 7),
          (0, 2), (1, 3), (4, 6), (5, 7),
          (1, 2), (5, 6), (0, 4), (3, 7),
          (1, 5), (2, 6),
          (1, 4), (3, 6),
          (2, 4), (3, 5),
          (3, 4)]


def _fused_body(q_ref, k_ref, v_ref, sal_ref, w1_ref, b1_ref, w2_ref, b2_ref,
                o_ref, x_scr, rw_scr, st_scr, *, S_b, NS, L, inv_sqrt_d):
    l = pl.program_id(1)
    p = pl.program_id(2)
    ns = pl.program_id(3)

    @pl.when(jnp.logical_and(jnp.logical_and(l == 0, p == 0), ns == 0))
    def _router():
        h = jax.lax.dot_general(q_ref[...], w1_ref[...], (((1,), (0,)), ((), ())),
                                preferred_element_type=jnp.float32)
        h = jnp.maximum(h + b1_ref[...], 0.0)
        logits = jax.lax.dot_general(h.astype(jnp.bfloat16), w2_ref[...],
                                     (((1,), (0,)), ((), ())),
                                     preferred_element_type=jnp.float32)
        logits = logits + b2_ref[...]
        mx = jnp.max(logits, axis=1, keepdims=True)
        e = jnp.exp(logits - mx)
        rw_scr[...] = e / jnp.sum(e, axis=1, keepdims=True)

    @pl.when(jnp.logical_and(p == 0, ns < NS))
    def _scores():
        s = jax.lax.dot_general(q_ref[...], k_ref[0], (((1,), (1,)), ((), ())),
                                preferred_element_type=jnp.float32)
        x_scr[:, pl.ds(ns * S_b, S_b)] = s + sal_ref[0, 0, pl.ds(ns * S_b, S_b)][None, :]

    @pl.when(jnp.logical_and(p == 1, ns == 0))
    def _topk_stats():
        def resort(c):
            n = len(c)
            if n == 1:
                return c
            h = n // 2
            hi = [jnp.maximum(c[i], c[i + h]) for i in range(h)]
            lo = [jnp.minimum(c[i], c[i + h]) for i in range(h)]
            return resort(hi) + resort(lo)

        Wseg = (S_b * NS) // 8
        s8 = [x_scr[:, j * Wseg:(j + 1) * Wseg] for j in range(8)]
        for i, j in _SORT8:
            hi = jnp.maximum(s8[i], s8[j])
            s8[j] = jnp.minimum(s8[i], s8[j])
            s8[i] = hi
        w_cur = Wseg
        while w_cur > 1:
            h = w_cur // 2
            s8 = [jnp.maximum(s8[i][:, :h], s8[7 - i][:, h:]) for i in range(8)]
            if h > 1:
                s8 = resort(s8)
            w_cur = h
        t0 = s8[0]
        theta = s8[0]
        for i in range(1, 8):
            t0 = jnp.maximum(t0, s8[i])
            theta = jnp.minimum(theta, s8[i])
        denom = jnp.zeros_like(t0)
        for i in range(8):
            denom = denom + jnp.exp((s8[i] - t0) * inv_sqrt_d)
        lane = jax.lax.broadcasted_iota(jnp.int32, (1, L), 1)
        rw_l = jnp.sum(rw_scr[...] * (lane == l).astype(jnp.float32), axis=1,
                       keepdims=True)
        st_scr[:, 0:1] = t0 - jnp.log(rw_l / denom) / inv_sqrt_d
        st_scr[:, 1:2] = theta

    @pl.when(p == 1)
    def _read():
        xt = x_scr[:, pl.ds(ns * S_b, S_b)]
        m = st_scr[:, 0:1]
        theta = st_scr[:, 1:2]
        w = jnp.where(xt >= theta, jnp.exp((xt - m) * inv_sqrt_d), 0.0)
        r = jax.lax.dot_general(w.astype(jnp.bfloat16), v_ref[0],
                                (((1,), (0,)), ((), ())),
                                preferred_element_type=jnp.float32)

        @pl.when(jnp.logical_and(l == 0, ns == 0))
        def _set():
            o_ref[...] = r

        @pl.when(jnp.logical_not(jnp.logical_and(l == 0, ns == 0)))
        def _acc():
            o_ref[...] += r


def kernel(q, K, V, salience, W1, b1, W2, b2):
    B, T, D = q.shape
    L, S, _ = K.shape
    H = W1.shape[1]
    BT = B * T
    TM = 1024 if BT % 1024 == 0 else BT
    S_b = 2048 if S % 2048 == 0 else S
    NS = S // S_b

    qb = q.reshape(BT, D).astype(jnp.bfloat16)
    Kb = K.astype(jnp.bfloat16)
    Vb = V.astype(jnp.bfloat16)
    sal3 = (salience * math.sqrt(D)).reshape(L, 1, S)
    W1b = W1.astype(jnp.bfloat16)
    W2b = W2.astype(jnp.bfloat16)
    b1r = b1.reshape(1, H)
    b2r = b2.reshape(1, L)

    body = functools.partial(_fused_body, S_b=S_b, NS=NS, L=L,
                             inv_sqrt_d=float(1.0 / math.sqrt(D)))
    out = pl.pallas_call(
        body,
        grid=(BT // TM, L, 2, NS),
        in_specs=[
            pl.BlockSpec((TM, D), lambda bt, l, p, ns: (bt, 0)),
            pl.BlockSpec((1, S_b, D),
                         lambda bt, l, p, ns: (l, jnp.where(p == 0, ns, NS - 1), 0)),
            pl.BlockSpec((1, S_b, D),
                         lambda bt, l, p, ns: (l, jnp.where(p == 1, ns, 0), 0)),
            pl.BlockSpec((1, 1, S), lambda bt, l, p, ns: (l, 0, 0)),
            pl.BlockSpec((D, H), lambda bt, l, p, ns: (0, 0)),
            pl.BlockSpec((1, H), lambda bt, l, p, ns: (0, 0)),
            pl.BlockSpec((H, L), lambda bt, l, p, ns: (0, 0)),
            pl.BlockSpec((1, L), lambda bt, l, p, ns: (0, 0)),
        ],
        out_specs=pl.BlockSpec((TM, D), lambda bt, l, p, ns: (bt, 0)),
        out_shape=jax.ShapeDtypeStruct((BT, D), jnp.float32),
        scratch_shapes=[
            pltpu.VMEM((TM, S), jnp.float32),
            pltpu.VMEM((TM, L), jnp.float32),
            pltpu.VMEM((TM, 8), jnp.float32),
        ],
        compiler_params=pltpu.CompilerParams(
            dimension_semantics=("arbitrary",) * 4,
            vmem_limit_bytes=66584576,
        ),
    )(qb, Kb, Vb, sal3, W1b, b1r, W2b, b2r)
    return out.reshape(B, T, D)


# speedup vs baseline: 1.1946x; 1.0264x over previous
import functools
import math

import jax
import jax.numpy as jnp
from jax.experimental import pallas as pl
from jax.experimental.pallas import tpu as pltpu

_SORT8 = [(0, 1), (2, 3), (4, 5), (6, 7),
          (0, 2), (1, 3), (4, 6), (5, 7),
          (1, 2), (5, 6), (0, 4), (3, 7),
          (1, 5), (2, 6),
          (1, 4), (3, 6),
          (2, 4), (3, 5),
          (3, 4)]


def _fused_body(q_ref, k_ref, v_ref, sal_ref, w1_ref, b1_ref, w2_ref, b2_ref,
                o_ref, x_scr, rw_scr, st_scr, *, S_b, NS, L, inv_sqrt_d):
    l = pl.program_id(1)
    p = pl.program_id(2)
    ns = pl.program_id(3)

    @pl.when(jnp.logical_and(jnp.logical_and(l == 0, p == 0), ns == 0))
    def _router():
        h = jax.lax.dot_general(q_ref[...], w1_ref[...], (((1,), (0,)), ((), ())),
                                preferred_element_type=jnp.float32)
        h = jnp.maximum(h + b1_ref[...], 0.0)
        logits = jax.lax.dot_general(h.astype(jnp.bfloat16), w2_ref[...],
                                     (((1,), (0,)), ((), ())),
                                     preferred_element_type=jnp.float32)
        logits = logits + b2_ref[...]
        mx = jnp.max(logits, axis=1, keepdims=True)
        e = jnp.exp(logits - mx)
        rw_scr[...] = e / jnp.sum(e, axis=1, keepdims=True)

    @pl.when(jnp.logical_and(p == 0, ns < NS))
    def _scores():
        s = jax.lax.dot_general(q_ref[...], k_ref[0], (((1,), (1,)), ((), ())),
                                preferred_element_type=jnp.float32)
        x_scr[:, pl.ds(ns * S_b, S_b)] = s + sal_ref[0, 0, pl.ds(ns * S_b, S_b)][None, :]

    @pl.when(jnp.logical_and(p == 1, ns == 0))
    def _topk_stats():
        def resort(c):
            n = len(c)
            if n == 1:
                return c
            h = n // 2
            hi = [jnp.maximum(c[i], c[i + h]) for i in range(h)]
            lo = [jnp.minimum(c[i], c[i + h]) for i in range(h)]
            return resort(hi) + resort(lo)

        Wseg = (S_b * NS) // 8
        s8 = [x_scr[:, j * Wseg:(j + 1) * Wseg] for j in range(8)]
        for i, j in _SORT8:
            hi = jnp.maximum(s8[i], s8[j])
            s8[j] = jnp.minimum(s8[i], s8[j])
            s8[i] = hi
        w_cur = Wseg
        while w_cur > 1:
            h = w_cur // 2
            s8 = [jnp.maximum(s8[i][:, :h], s8[7 - i][:, h:]) for i in range(8)]
            if h > 1:
                s8 = resort(s8)
            w_cur = h
        t0 = s8[0]
        theta = s8[0]
        for i in range(1, 8):
            t0 = jnp.maximum(t0, s8[i])
            theta = jnp.minimum(theta, s8[i])
        c2 = inv_sqrt_d * 1.4426950408889634
        denom = jnp.zeros_like(t0)
        for i in range(8):
            denom = denom + jnp.exp2((s8[i] - t0) * c2)
        lane = jax.lax.broadcasted_iota(jnp.int32, (1, L), 1)
        rw_l = jnp.sum(rw_scr[...] * (lane == l).astype(jnp.float32), axis=1,
                       keepdims=True)
        st_scr[:, 0:1] = t0 - jnp.log2(rw_l / denom) / c2
        st_scr[:, 1:2] = theta

    @pl.when(p == 1)
    def _read():
        c2 = inv_sqrt_d * 1.4426950408889634
        m = st_scr[:, 0:1]
        theta = st_scr[:, 1:2]
        CH = 4
        Wc = S_b // CH
        r = None
        for c in range(CH):
            xt = x_scr[:, pl.ds(ns * S_b + c * Wc, Wc)]
            w = jnp.where(xt >= theta, jnp.exp2((xt - m) * c2), 0.0)
            rc = jax.lax.dot_general(w.astype(jnp.bfloat16),
                                     v_ref[0, pl.ds(c * Wc, Wc), :],
                                     (((1,), (0,)), ((), ())),
                                     preferred_element_type=jnp.float32)
            r = rc if r is None else r + rc

        @pl.when(jnp.logical_and(l == 0, ns == 0))
        def _set():
            o_ref[...] = r

        @pl.when(jnp.logical_not(jnp.logical_and(l == 0, ns == 0)))
        def _acc():
            o_ref[...] += r


def kernel(q, K, V, salience, W1, b1, W2, b2):
    B, T, D = q.shape
    L, S, _ = K.shape
    H = W1.shape[1]
    BT = B * T
    TM = 1024 if BT % 1024 == 0 else BT
    S_b = 2048 if S % 2048 == 0 else S
    NS = S // S_b

    qb = q.reshape(BT, D).astype(jnp.bfloat16)
    Kb = K.astype(jnp.bfloat16)
    Vb = V.astype(jnp.bfloat16)
    sal3 = (salience * math.sqrt(D)).reshape(L, 1, S)
    W1b = W1.astype(jnp.bfloat16)
    W2b = W2.astype(jnp.bfloat16)
    b1r = b1.reshape(1, H)
    b2r = b2.reshape(1, L)

    body = functools.partial(_fused_body, S_b=S_b, NS=NS, L=L,
                             inv_sqrt_d=float(1.0 / math.sqrt(D)))
    out = pl.pallas_call(
        body,
        grid=(BT // TM, L, 2, NS),
        in_specs=[
            pl.BlockSpec((TM, D), lambda bt, l, p, ns: (bt, 0)),
            pl.BlockSpec((1, S_b, D),
                         lambda bt, l, p, ns: (l, jnp.where(p == 0, ns, NS - 1), 0)),
            pl.BlockSpec((1, S_b, D),
                         lambda bt, l, p, ns: (l, jnp.where(p == 1, ns, 0), 0)),
            pl.BlockSpec((1, 1, S), lambda bt, l, p, ns: (l, 0, 0)),
            pl.BlockSpec((D, H), lambda bt, l, p, ns: (0, 0)),
            pl.BlockSpec((1, H), lambda bt, l, p, ns: (0, 0)),
            pl.BlockSpec((H, L), lambda bt, l, p, ns: (0, 0)),
            pl.BlockSpec((1, L), lambda bt, l, p, ns: (0, 0)),
        ],
        out_specs=pl.BlockSpec((TM, D), lambda bt, l, p, ns: (bt, 0)),
        out_shape=jax.ShapeDtypeStruct((BT, D), jnp.float32),
        scratch_shapes=[
            pltpu.VMEM((TM, S), jnp.float32),
            pltpu.VMEM((TM, L), jnp.float32),
            pltpu.VMEM((TM, 8), jnp.float32),
        ],
        compiler_params=pltpu.CompilerParams(
            dimension_semantics=("arbitrary",) * 4,
            vmem_limit_bytes=66584576,
        ),
    )(qb, Kb, Vb, sal3, W1b, b1r, W2b, b2r)
    return out.reshape(B, T, D)
